# Initial kernel scaffold; baseline (speedup 1.0000x reference)
import jax
import jax.numpy as jnp
from jax import lax
from jax.experimental import pallas as pl
from jax.experimental.pallas import tpu as pltpu
from jax.experimental.pallas import tpu_sc as plsc

N = 10000
E = 320000
F = 128
NC = 2
NS = 16
NW = NC * NS
EPW = E // NW
K = 80
NCHUNK = EPW // K
RPT = N // NS
CW = 16
BN = 1000


def _make_agg(with_cnt):
  outs = [jax.ShapeDtypeStruct((NC, N, F), jnp.float32)]
  scratch = [
      pltpu.VMEM_SHARED((N, F), jnp.float32),
      pltpu.VMEM((RPT, F), jnp.float32),
      pltpu.VMEM((K, F), jnp.float32),
      pltpu.VMEM((K,), jnp.int32),
      pltpu.VMEM((K,), jnp.int32),
      pltpu.VMEM((K,), jnp.float32),
      pltpu.SemaphoreType.DMA,
  ]
  if with_cnt:
    outs.append(jax.ShapeDtypeStruct((NC, N, CW), jnp.float32))
    scratch += [
        pltpu.VMEM_SHARED((N, CW), jnp.float32),
        pltpu.VMEM((RPT, CW), jnp.float32),
        pltpu.VMEM((K, CW), jnp.float32),
    ]
  mesh = plsc.VectorSubcoreMesh(
      core_axis_name="c", subcore_axis_name="s", num_cores=NC, num_subcores=NS)

  def body(*refs):
    if with_cnt:
      (x_hbm, src_hbm, dst_hbm, w_hbm, z_hbm, zc_hbm, ones_hbm,
       acc_out, cnt_out,
       acc_sh, vbuf, rows, sidx, didx, wbuf, sem, cnt_sh, vcnt, onesb) = refs
    else:
      (x_hbm, src_hbm, dst_hbm, w_hbm, z_hbm,
       acc_out,
       acc_sh, vbuf, rows, sidx, didx, wbuf, sem) = refs
    c = lax.axis_index("c")
    s = lax.axis_index("s")
    wid = c * NS + s
    myrows = pl.ds(s * RPT, RPT)

    pltpu.sync_copy(z_hbm, vbuf)
    pltpu.sync_copy(vbuf, acc_sh.at[myrows])
    if with_cnt:
      pltpu.sync_copy(zc_hbm, vcnt)
      pltpu.sync_copy(vcnt, cnt_sh.at[myrows])
      pltpu.sync_copy(ones_hbm, onesb)
    plsc.subcore_barrier()

    def chunk(i, carry):
      base = wid * EPW + i * K
      pltpu.sync_copy(src_hbm.at[pl.ds(base, K)], sidx)
      pltpu.sync_copy(dst_hbm.at[pl.ds(base, K)], didx)
      pltpu.sync_copy(w_hbm.at[pl.ds(base, K)], wbuf)
      pltpu.async_copy(x_hbm.at[sidx], rows, sem).wait()
      for g in range(K // 16):
        for j in range(16):
          k = g * 16 + j
          wb = plsc.load_gather(wbuf, [jnp.full((16,), k, jnp.int32)])
          for f in range(F // 16):
            sl = pl.ds(f * 16, 16)
            rows[k, sl] = rows[k, sl] * wb
      pltpu.sync_copy(rows, acc_sh.at[didx], add=True)
      if with_cnt:
        pltpu.sync_copy(onesb, cnt_sh.at[didx], add=True)
      return carry

    lax.fori_loop(0, NCHUNK, chunk, 0)
    plsc.subcore_barrier()

    pltpu.sync_copy(acc_sh.at[myrows], vbuf)
    pltpu.sync_copy(vbuf, acc_out.at[c, myrows])
    if with_cnt:
      pltpu.sync_copy(cnt_sh.at[myrows], vcnt)
      pltpu.sync_copy(vcnt, cnt_out.at[c, myrows])

  return pl.kernel(body, out_type=tuple(outs), mesh=mesh,
                   scratch_types=scratch)


_agg_cnt = _make_agg(True)
_agg = _make_agg(False)

_DOT = dict(preferred_element_type=jnp.float32, precision=lax.Precision.HIGHEST)


def _dense0_body(x, a0, a1, c0, c1, wr, wt, br, bt, wro, bro, out):
  cnt = c0[:, 0:1] + c1[:, 0:1]
  mean = (a0[...] + a1[...]) / jnp.maximum(cnt, 1.0)
  t = (jnp.dot(mean, wr[...], **_DOT) + jnp.dot(x[...], wt[...], **_DOT)
       + br[...] + bt[...])
  hc = jnp.maximum(t, 0.0)
  out[...] = jnp.maximum(jnp.dot(hc, wro[...], **_DOT) + bro[...], 0.0)


def _dense1_body(h, a0, a1, c0, c1, wr, wt, br, bt, wro, bro, wp, bp, out):
  cnt = c0[:, 0:1] + c1[:, 0:1]
  mean = (a0[...] + a1[...]) / jnp.maximum(cnt, 1.0)
  t = (jnp.dot(mean, wr[...], **_DOT) + jnp.dot(h[...], wt[...], **_DOT)
       + br[...] + bt[...])
  hc = jnp.maximum(t, 0.0)
  r = jnp.maximum(jnp.dot(hc, wro[...], **_DOT) + bro[...], 0.0)
  out[...] = jnp.dot(r, wp[...], **_DOT) + bp[...]


_row = pl.BlockSpec((BN, F), lambda i: (i, 0))
_cntb = pl.BlockSpec((BN, CW), lambda i: (i, 0))
_wmat = pl.BlockSpec((F, F), lambda i: (0, 0))
_bvec = pl.BlockSpec((1, F), lambda i: (0, 0))

_dense0 = pl.pallas_call(
    _dense0_body,
    grid=(N // BN,),
    in_specs=[_row, _row, _row, _cntb, _cntb, _wmat, _wmat, _bvec, _bvec,
              _wmat, _bvec],
    out_specs=_row,
    out_shape=jax.ShapeDtypeStruct((N, F), jnp.float32),
)

_dense1 = pl.pallas_call(
    _dense1_body,
    grid=(N // BN,),
    in_specs=[_row, _row, _row, _cntb, _cntb, _wmat, _wmat, _bvec, _bvec,
              _wmat, _bvec, _wmat, _bvec],
    out_specs=_row,
    out_shape=jax.ShapeDtypeStruct((N, F), jnp.float32),
)


def kernel(dynamic_node_features, edge_index, edge_weight,
           W_rel0, b_rel0, W_root0, b_root0, W_ro0, b_ro0,
           W_rel1, b_rel1, W_root1, b_root1, W_ro1, b_ro1,
           W_prd, b_prd):
  x = dynamic_node_features[:, 0, :]
  src = edge_index[0]
  dst = edge_index[1]
  zrow = jnp.zeros((RPT, F), jnp.float32)
  zcnt = jnp.zeros((RPT, CW), jnp.float32)
  onesb = jnp.ones((K, CW), jnp.float32)

  acc_p, cnt_p = _agg_cnt(x, src, dst, edge_weight, zrow, zcnt, onesb)
  h = _dense0(x, acc_p[0], acc_p[1], cnt_p[0], cnt_p[1],
              W_rel0.T, W_root0.T, b_rel0.reshape(1, F), b_root0.reshape(1, F),
              W_ro0.T, b_ro0.reshape(1, F))
  acc1_p = _agg(h, src, dst, edge_weight, zrow)
  wp_pad = jnp.pad(W_prd.T, ((0, 0), (0, F - 1)))
  bp_pad = jnp.pad(b_prd.reshape(1, 1), ((0, 0), (0, F - 1)))
  xh = _dense1(h, acc1_p[0], acc1_p[1], cnt_p[0], cnt_p[1],
               W_rel1.T, W_root1.T, b_rel1.reshape(1, F), b_root1.reshape(1, F),
               W_ro1.T, b_ro1.reshape(1, F), wp_pad, bp_pad)
  return xh[:, :1].reshape(N, 1, 1)


# speedup vs baseline: 3.1959x; 3.1959x over previous
import jax
import jax.numpy as jnp
from jax import lax
from jax.experimental import pallas as pl
from jax.experimental.pallas import tpu as pltpu
from jax.experimental.pallas import tpu_sc as plsc

N = 10000
E = 320000
F = 128
NC = 2
NS = 16
NW = NC * NS
EPW = E // NW
K = 80
NCHUNK = EPW // K
NP = 10240
RPT = NP // NS
ZB = 128
NZB = RPT // ZB
CW = 16
BN = 1000

_MESH = plsc.VectorSubcoreMesh(
    core_axis_name="c", subcore_axis_name="s", num_cores=NC, num_subcores=NS)
_PARAMS = pltpu.CompilerParams(needs_layout_passes=False)


def _agg_body(x_hbm, src_hbm, dst_hbm, w_hbm, z_hbm, iota_hbm, bidx_hbm,
              acc_out,
              acc_sh, vbuf, idxmat, rows, sidx, didx, wbuf, bidx, gsem, ssem):
  c = lax.axis_index("c")
  s = lax.axis_index("s")
  wid = c * NS + s
  pltpu.async_copy(iota_hbm.at[s], idxmat, ssem).wait()
  pltpu.async_copy(bidx_hbm, bidx, ssem).wait()
  pltpu.async_copy(z_hbm, vbuf, ssem).wait()

  for j in range(NZB):
    pltpu.async_copy(vbuf, acc_sh.at[idxmat.at[j]], ssem).wait()
  plsc.subcore_barrier()

  def chunk(i, carry):
    base = wid * EPW + i * K
    pltpu.async_copy(src_hbm.at[pl.ds(base, K)], sidx, ssem).wait()
    pltpu.async_copy(dst_hbm.at[pl.ds(base, K)], didx, ssem).wait()
    pltpu.async_copy(w_hbm.at[pl.ds(base, K)], wbuf, ssem).wait()
    pltpu.async_copy(x_hbm.at[sidx], rows, gsem).wait()
    for g in range(K // 16):
      for j in range(16):
        k = g * 16 + j
        wb = plsc.load_gather(wbuf, [bidx[j, :] + (g * 16)])
        for f in range(F // 16):
          sl = pl.ds(f * 16, 16)
          rows[k, sl] = rows[k, sl] * wb
    pltpu.async_copy(rows, acc_sh.at[didx], ssem, add=True).wait()
    return carry

  lax.fori_loop(0, NCHUNK, chunk, 0)
  plsc.subcore_barrier()

  for j in range(NZB):
    pltpu.async_copy(acc_sh.at[idxmat.at[j]], vbuf, gsem).wait()
    pltpu.async_copy(vbuf, acc_out.at[c, pl.ds(s * RPT + j * ZB, ZB)],
                     ssem).wait()


_agg = pl.kernel(
    _agg_body,
    out_type=jax.ShapeDtypeStruct((NC, NP, F), jnp.float32),
    mesh=_MESH,
    scratch_types=[
        pltpu.VMEM_SHARED((NP, F), jnp.float32),
        pltpu.VMEM((ZB, F), jnp.float32),
        pltpu.VMEM((NZB, ZB), jnp.int32),
        pltpu.VMEM((K, F), jnp.float32),
        pltpu.VMEM((K,), jnp.int32),
        pltpu.VMEM((K,), jnp.int32),
        pltpu.VMEM((K,), jnp.float32),
        pltpu.VMEM((16, 16), jnp.int32),
        pltpu.SemaphoreType.DMA,
        pltpu.SemaphoreType.DMA,
    ],
    compiler_params=_PARAMS,
)


def _cnt_body(dst_hbm, zc_hbm, iota_hbm, ones_hbm,
              cnt_out,
              cnt_sh, vcnt, idxmat, didx, onesb, gsem, ssem):
  c = lax.axis_index("c")
  s = lax.axis_index("s")
  wid = c * NS + s

  pltpu.async_copy(iota_hbm.at[s], idxmat, ssem).wait()
  pltpu.async_copy(zc_hbm, vcnt, ssem).wait()
  pltpu.async_copy(ones_hbm, onesb, ssem).wait()

  for j in range(NZB):
    pltpu.async_copy(vcnt, cnt_sh.at[idxmat.at[j]], ssem).wait()
  plsc.subcore_barrier()

  def chunk(i, carry):
    base = wid * EPW + i * K
    pltpu.async_copy(dst_hbm.at[pl.ds(base, K)], didx, ssem).wait()
    pltpu.async_copy(onesb, cnt_sh.at[didx], ssem, add=True).wait()
    return carry

  lax.fori_loop(0, NCHUNK, chunk, 0)
  plsc.subcore_barrier()

  for j in range(NZB):
    pltpu.async_copy(cnt_sh.at[idxmat.at[j]], vcnt, gsem).wait()
    pltpu.async_copy(vcnt, cnt_out.at[c, pl.ds(s * RPT + j * ZB, ZB)],
                     ssem).wait()


_cnt = pl.kernel(
    _cnt_body,
    out_type=jax.ShapeDtypeStruct((NC, NP, F), jnp.float32),
    mesh=_MESH,
    scratch_types=[
        pltpu.VMEM_SHARED((NP, F), jnp.float32),
        pltpu.VMEM((ZB, F), jnp.float32),
        pltpu.VMEM((NZB, ZB), jnp.int32),
        pltpu.VMEM((K,), jnp.int32),
        pltpu.VMEM((K, F), jnp.float32),
        pltpu.SemaphoreType.DMA,
        pltpu.SemaphoreType.DMA,
    ],
    compiler_params=_PARAMS,
)

_DOT = dict(preferred_element_type=jnp.float32, precision=lax.Precision.HIGHEST)


def _dense0_body(x, a0, a1, c0, c1, wr, wt, br, bt, wro, bro, out):
  cnt = c0[:, 0:1] + c1[:, 0:1]
  mean = (a0[...] + a1[...]) / jnp.maximum(cnt, 1.0)
  t = (jnp.dot(mean, wr[...], **_DOT) + jnp.dot(x[...], wt[...], **_DOT)
       + br[...] + bt[...])
  hc = jnp.maximum(t, 0.0)
  out[...] = jnp.maximum(jnp.dot(hc, wro[...], **_DOT) + bro[...], 0.0)


def _dense1_body(h, a0, a1, c0, c1, wr, wt, br, bt, wro, bro, wp, bp, out):
  cnt = c0[:, 0:1] + c1[:, 0:1]
  mean = (a0[...] + a1[...]) / jnp.maximum(cnt, 1.0)
  t = (jnp.dot(mean, wr[...], **_DOT) + jnp.dot(h[...], wt[...], **_DOT)
       + br[...] + bt[...])
  hc = jnp.maximum(t, 0.0)
  r = jnp.maximum(jnp.dot(hc, wro[...], **_DOT) + bro[...], 0.0)
  out[...] = jnp.dot(r, wp[...], **_DOT) + bp[...]


_row = pl.BlockSpec((BN, F), lambda i: (i, 0))
_cntb = pl.BlockSpec((BN, CW), lambda i: (i, 0))
_wmat = pl.BlockSpec((F, F), lambda i: (0, 0))
_bvec = pl.BlockSpec((1, F), lambda i: (0, 0))

_dense0 = pl.pallas_call(
    _dense0_body,
    grid=(N // BN,),
    in_specs=[_row, _row, _row, _cntb, _cntb, _wmat, _wmat, _bvec, _bvec,
              _wmat, _bvec],
    out_specs=_row,
    out_shape=jax.ShapeDtypeStruct((N, F), jnp.float32),
)

_dense1 = pl.pallas_call(
    _dense1_body,
    grid=(N // BN,),
    in_specs=[_row, _row, _row, _cntb, _cntb, _wmat, _wmat, _bvec, _bvec,
              _wmat, _bvec, _wmat, _bvec],
    out_specs=_row,
    out_shape=jax.ShapeDtypeStruct((N, F), jnp.float32),
)


def kernel(dynamic_node_features, edge_index, edge_weight,
           W_rel0, b_rel0, W_root0, b_root0, W_ro0, b_ro0,
           W_rel1, b_rel1, W_root1, b_root1, W_ro1, b_ro1,
           W_prd, b_prd):
  x = dynamic_node_features[:, 0, :]
  src = edge_index[0]
  dst = edge_index[1]
  ew = edge_weight
  zrow = jnp.zeros((ZB, F), jnp.float32)
  onesb = jnp.ones((K, F), jnp.float32)
  iota3 = jnp.arange(NP, dtype=jnp.int32).reshape(NS, NZB, ZB)

  bidx = jnp.broadcast_to(jnp.arange(16, dtype=jnp.int32)[:, None], (16, 16))
  cnt_p = _cnt(dst, zrow, iota3, onesb)
  acc_p = _agg(x, src, dst, ew, zrow, iota3, bidx)
  c0, c1 = cnt_p[0, :N, :CW], cnt_p[1, :N, :CW]
  h = _dense0(x, acc_p[0, :N], acc_p[1, :N], c0, c1,
              W_rel0.T, W_root0.T, b_rel0.reshape(1, F), b_root0.reshape(1, F),
              W_ro0.T, b_ro0.reshape(1, F))
  acc1_p = _agg(h, src, dst, ew, zrow, iota3, bidx)
  wp_pad = jnp.pad(W_prd.T, ((0, 0), (0, F - 1)))
  bp_pad = jnp.pad(b_prd.reshape(1, 1), ((0, 0), (0, F - 1)))
  xh = _dense1(h, acc1_p[0, :N], acc1_p[1, :N], c0, c1,
               W_rel1.T, W_root1.T, b_rel1.reshape(1, F), b_root1.reshape(1, F),
               W_ro1.T, b_ro1.reshape(1, F), wp_pad, bp_pad)
  return xh[:, :1].reshape(N, 1, 1)


# speedup vs baseline: 3.3800x; 1.0576x over previous
import jax
import jax.numpy as jnp
from jax import lax
from jax.experimental import pallas as pl
from jax.experimental.pallas import tpu as pltpu
from jax.experimental.pallas import tpu_sc as plsc

N = 10000
E = 320000
F = 128
NC = 2
NS = 16
NW = NC * NS
EPW = E // NW
K = 80
NCHUNK = EPW // K
NP = 10240
RPT = NP // NS
ZB = 128
NZB = RPT // ZB
CW = 16
BN = 1000

_MESH = plsc.VectorSubcoreMesh(
    core_axis_name="c", subcore_axis_name="s", num_cores=NC, num_subcores=NS)
_PARAMS = pltpu.CompilerParams(needs_layout_passes=False)


def _agg_body(x_hbm, src_hbm, dst_hbm, w_hbm, z_hbm, iota_hbm, bidx_hbm,
              acc_out,
              acc_sh, vbuf, idxmat, rows, sidx, didx, wbuf, bidx, gsem, ssem):
  c = lax.axis_index("c")
  s = lax.axis_index("s")
  wid = c * NS + s
  pltpu.async_copy(iota_hbm.at[s], idxmat, ssem).wait()
  pltpu.async_copy(bidx_hbm, bidx, ssem).wait()
  pltpu.async_copy(z_hbm, vbuf, ssem).wait()

  for j in range(NZB):
    pltpu.async_copy(vbuf, acc_sh.at[idxmat.at[j]], ssem).wait()
  plsc.subcore_barrier()

  def chunk(i, carry):
    base = wid * EPW + i * K
    pltpu.async_copy(src_hbm.at[pl.ds(base, K)], sidx, ssem).wait()
    pltpu.async_copy(dst_hbm.at[pl.ds(base, K)], didx, ssem).wait()
    pltpu.async_copy(w_hbm.at[pl.ds(base, K)], wbuf, ssem).wait()
    pltpu.async_copy(x_hbm.at[sidx], rows, gsem).wait()
    for g in range(K // 16):
      for j in range(16):
        k = g * 16 + j
        wb = plsc.load_gather(wbuf, [bidx[j, :] + (g * 16)])
        for f in range(F // 16):
          sl = pl.ds(f * 16, 16)
          rows[k, sl] = rows[k, sl] * wb
    pltpu.async_copy(rows, acc_sh.at[didx], ssem, add=True).wait()
    return carry

  lax.fori_loop(0, NCHUNK, chunk, 0)
  plsc.subcore_barrier()

  for j in range(NZB):
    pltpu.async_copy(acc_sh.at[idxmat.at[j]], vbuf, gsem).wait()
    pltpu.async_copy(vbuf, acc_out.at[c, pl.ds(s * RPT + j * ZB, ZB)],
                     ssem).wait()


_agg = pl.kernel(
    _agg_body,
    out_type=jax.ShapeDtypeStruct((NC, NP, F), jnp.float32),
    mesh=_MESH,
    scratch_types=[
        pltpu.VMEM_SHARED((NP, F), jnp.float32),
        pltpu.VMEM((ZB, F), jnp.float32),
        pltpu.VMEM((NZB, ZB), jnp.int32),
        pltpu.VMEM((K, F), jnp.float32),
        pltpu.VMEM((K,), jnp.int32),
        pltpu.VMEM((K,), jnp.int32),
        pltpu.VMEM((K,), jnp.float32),
        pltpu.VMEM((16, 16), jnp.int32),
        pltpu.SemaphoreType.DMA,
        pltpu.SemaphoreType.DMA,
    ],
    compiler_params=_PARAMS,
)


def _cnt_body(dst_hbm, zc_hbm, iota_hbm, ones_hbm,
              cnt_out,
              cnt_sh, vcnt, idxmat, didx, onesb, gsem, ssem):
  c = lax.axis_index("c")
  s = lax.axis_index("s")
  wid = c * NS + s

  pltpu.async_copy(iota_hbm.at[s], idxmat, ssem).wait()
  pltpu.async_copy(zc_hbm, vcnt, ssem).wait()
  pltpu.async_copy(ones_hbm, onesb, ssem).wait()

  for j in range(NZB):
    pltpu.async_copy(vcnt, cnt_sh.at[idxmat.at[j]], ssem).wait()
  plsc.subcore_barrier()

  def chunk(i, carry):
    base = wid * EPW + i * K
    pltpu.async_copy(dst_hbm.at[pl.ds(base, K)], didx, ssem).wait()
    pltpu.async_copy(onesb, cnt_sh.at[didx], ssem, add=True).wait()
    return carry

  lax.fori_loop(0, NCHUNK, chunk, 0)
  plsc.subcore_barrier()

  for j in range(NZB):
    pltpu.async_copy(cnt_sh.at[idxmat.at[j]], vcnt, gsem).wait()
    pltpu.async_copy(vcnt, cnt_out.at[c, pl.ds(s * RPT + j * ZB, ZB)],
                     ssem).wait()


_cnt = pl.kernel(
    _cnt_body,
    out_type=jax.ShapeDtypeStruct((NC, NP, F), jnp.float32),
    mesh=_MESH,
    scratch_types=[
        pltpu.VMEM_SHARED((NP, F), jnp.float32),
        pltpu.VMEM((ZB, F), jnp.float32),
        pltpu.VMEM((NZB, ZB), jnp.int32),
        pltpu.VMEM((K,), jnp.int32),
        pltpu.VMEM((K, F), jnp.float32),
        pltpu.SemaphoreType.DMA,
        pltpu.SemaphoreType.DMA,
    ],
    compiler_params=_PARAMS,
)

_DOT = dict(preferred_element_type=jnp.float32)


def _dense0_body(x, a0, a1, c0, c1, wr, wt, br, bt, wro, bro, out):
  cnt = c0[:, 0:1] + c1[:, 0:1]
  mean = (a0[...] + a1[...]) / jnp.maximum(cnt, 1.0)
  t = (jnp.dot(mean, wr[...], **_DOT) + jnp.dot(x[...], wt[...], **_DOT)
       + br[...] + bt[...])
  hc = jnp.maximum(t, 0.0)
  out[...] = jnp.maximum(jnp.dot(hc, wro[...], **_DOT) + bro[...], 0.0)


def _dense1_body(h, a0, a1, c0, c1, wr, wt, br, bt, wro, bro, wp, bp, out):
  cnt = c0[:, 0:1] + c1[:, 0:1]
  mean = (a0[...] + a1[...]) / jnp.maximum(cnt, 1.0)
  t = (jnp.dot(mean, wr[...], **_DOT) + jnp.dot(h[...], wt[...], **_DOT)
       + br[...] + bt[...])
  hc = jnp.maximum(t, 0.0)
  r = jnp.maximum(jnp.dot(hc, wro[...], **_DOT) + bro[...], 0.0)
  out[...] = jnp.dot(r, wp[...], **_DOT) + bp[...]


_row = pl.BlockSpec((BN, F), lambda i: (i, 0))
_cntb = pl.BlockSpec((BN, CW), lambda i: (i, 0))
_wmat = pl.BlockSpec((F, F), lambda i: (0, 0))
_bvec = pl.BlockSpec((1, F), lambda i: (0, 0))

_dense0 = pl.pallas_call(
    _dense0_body,
    grid=(N // BN,),
    in_specs=[_row, _row, _row, _cntb, _cntb, _wmat, _wmat, _bvec, _bvec,
              _wmat, _bvec],
    out_specs=_row,
    out_shape=jax.ShapeDtypeStruct((N, F), jnp.float32),
)

_dense1 = pl.pallas_call(
    _dense1_body,
    grid=(N // BN,),
    in_specs=[_row, _row, _row, _cntb, _cntb, _wmat, _wmat, _bvec, _bvec,
              _wmat, _bvec, _wmat, _bvec],
    out_specs=_row,
    out_shape=jax.ShapeDtypeStruct((N, F), jnp.float32),
)


def kernel(dynamic_node_features, edge_index, edge_weight,
           W_rel0, b_rel0, W_root0, b_root0, W_ro0, b_ro0,
           W_rel1, b_rel1, W_root1, b_root1, W_ro1, b_ro1,
           W_prd, b_prd):
  x = dynamic_node_features[:, 0, :]
  src = edge_index[0]
  dst = edge_index[1]
  ew = edge_weight
  zrow = jnp.zeros((ZB, F), jnp.float32)
  onesb = jnp.ones((K, F), jnp.float32)
  iota3 = jnp.arange(NP, dtype=jnp.int32).reshape(NS, NZB, ZB)

  bidx = jnp.broadcast_to(jnp.arange(16, dtype=jnp.int32)[:, None], (16, 16))
  cnt_p = _cnt(dst, zrow, iota3, onesb)
  acc_p = _agg(x, src, dst, ew, zrow, iota3, bidx)
  c0, c1 = cnt_p[0, :N, :CW], cnt_p[1, :N, :CW]
  h = _dense0(x, acc_p[0, :N], acc_p[1, :N], c0, c1,
              W_rel0.T, W_root0.T, b_rel0.reshape(1, F), b_root0.reshape(1, F),
              W_ro0.T, b_ro0.reshape(1, F))
  acc1_p = _agg(h, src, dst, ew, zrow, iota3, bidx)
  wp_pad = jnp.pad(W_prd.T, ((0, 0), (0, F - 1)))
  bp_pad = jnp.pad(b_prd.reshape(1, 1), ((0, 0), (0, F - 1)))
  xh = _dense1(h, acc1_p[0, :N], acc1_p[1, :N], c0, c1,
               W_rel1.T, W_root1.T, b_rel1.reshape(1, F), b_root1.reshape(1, F),
               W_ro1.T, b_ro1.reshape(1, F), wp_pad, bp_pad)
  return xh[:, :1].reshape(N, 1, 1)


# speedup vs baseline: 3.6640x; 1.0840x over previous
import jax
import jax.numpy as jnp
from jax import lax
from jax.experimental import pallas as pl
from jax.experimental.pallas import tpu as pltpu
from jax.experimental.pallas import tpu_sc as plsc

N = 10000
E = 320000
F = 128
NC = 2
NS = 16
NW = NC * NS
EPW = E // NW
K = 80
NCHUNK = EPW // K
NP = 10240
RPT = NP // NS
ZB = 128
NZB = RPT // ZB
CW = 16
BN = 1000

_MESH = plsc.VectorSubcoreMesh(
    core_axis_name="c", subcore_axis_name="s", num_cores=NC, num_subcores=NS)
_PARAMS = pltpu.CompilerParams(needs_layout_passes=False)


def _agg_body(x_hbm, src_hbm, dst_hbm, w_hbm, z_hbm, iota_hbm, bidx_hbm,
              acc_out,
              acc_sh, vbuf, idxmat, bidx,
              rows0, sidx0, didx0, wbuf0, rows1, sidx1, didx1, wbuf1,
              ssem, gsem0, gsem1, ssm0, ssm1,
              is00, is01, is02, is10, is11, is12):
  c = lax.axis_index("c")
  s = lax.axis_index("s")
  wid = c * NS + s
  ROWS = (rows0, rows1)
  SIDX = (sidx0, sidx1)
  DIDX = (didx0, didx1)
  WBUF = (wbuf0, wbuf1)
  GSEM = (gsem0, gsem1)
  SSEM = (ssm0, ssm1)
  ISEM = ((is00, is01, is02), (is10, is11, is12))

  pltpu.async_copy(iota_hbm.at[s], idxmat, ssem).wait()
  pltpu.async_copy(bidx_hbm, bidx, ssem).wait()
  pltpu.async_copy(z_hbm, vbuf, ssem).wait()

  for j in range(NZB):
    pltpu.async_copy(vbuf, acc_sh.at[idxmat.at[j]], ssem).wait()
  plsc.subcore_barrier()

  def scale_rows(rows_ref, wbuf_ref):
    for g in range(K // 16):
      for j in range(16):
        k = g * 16 + j
        wb = plsc.load_gather(wbuf_ref, [bidx[j, :] + (g * 16)])
        for f in range(F // 16):
          sl = pl.ds(f * 16, 16)
          rows_ref[k, sl] = rows_ref[k, sl] * wb

  def start_idx(ii, b):
    base = wid * EPW + ii * K
    return (pltpu.async_copy(src_hbm.at[pl.ds(base, K)], SIDX[b], ISEM[b][0]),
            pltpu.async_copy(dst_hbm.at[pl.ds(base, K)], DIDX[b], ISEM[b][1]),
            pltpu.async_copy(w_hbm.at[pl.ds(base, K)], WBUF[b], ISEM[b][2]))

  def chunk_pair(t, carry):
    ia = 2 * t
    descs = [start_idx(ia, 0), start_idx(ia + 1, 1)]
    gath = []
    for b in (0, 1):
      descs[b][0].wait()
      gath.append(pltpu.async_copy(x_hbm.at[SIDX[b]], ROWS[b], GSEM[b]))
    scat = []
    for b in (0, 1):
      descs[b][2].wait()
      gath[b].wait()
      scale_rows(ROWS[b], WBUF[b])
      descs[b][1].wait()
      scat.append(pltpu.async_copy(ROWS[b], acc_sh.at[DIDX[b]], SSEM[b],
                                   add=True))
    scat[0].wait()
    scat[1].wait()
    return carry

  lax.fori_loop(0, NCHUNK // 2, chunk_pair, 0)
  if NCHUNK % 2:
    d = start_idx(NCHUNK - 1, 0)
    d[0].wait()
    g = pltpu.async_copy(x_hbm.at[sidx0], rows0, gsem0)
    d[2].wait()
    g.wait()
    scale_rows(rows0, wbuf0)
    d[1].wait()
    pltpu.async_copy(rows0, acc_sh.at[didx0], ssm0, add=True).wait()
  plsc.subcore_barrier()

  for j in range(NZB):
    pltpu.async_copy(acc_sh.at[idxmat.at[j]], vbuf, gsem0).wait()
    pltpu.async_copy(vbuf, acc_out.at[c, pl.ds(s * RPT + j * ZB, ZB)],
                     ssem).wait()


_agg = pl.kernel(
    _agg_body,
    out_type=jax.ShapeDtypeStruct((NC, NP, F), jnp.float32),
    mesh=_MESH,
    scratch_types=[
        pltpu.VMEM_SHARED((NP, F), jnp.float32),
        pltpu.VMEM((ZB, F), jnp.float32),
        pltpu.VMEM((NZB, ZB), jnp.int32),
        pltpu.VMEM((16, 16), jnp.int32),
        pltpu.VMEM((K, F), jnp.float32),
        pltpu.VMEM((K,), jnp.int32),
        pltpu.VMEM((K,), jnp.int32),
        pltpu.VMEM((K,), jnp.float32),
        pltpu.VMEM((K, F), jnp.float32),
        pltpu.VMEM((K,), jnp.int32),
        pltpu.VMEM((K,), jnp.int32),
        pltpu.VMEM((K,), jnp.float32),
        pltpu.SemaphoreType.DMA,
        pltpu.SemaphoreType.DMA,
        pltpu.SemaphoreType.DMA,
        pltpu.SemaphoreType.DMA,
        pltpu.SemaphoreType.DMA,
        pltpu.SemaphoreType.DMA,
        pltpu.SemaphoreType.DMA,
        pltpu.SemaphoreType.DMA,
        pltpu.SemaphoreType.DMA,
        pltpu.SemaphoreType.DMA,
        pltpu.SemaphoreType.DMA,
    ],
    compiler_params=_PARAMS,
)


def _cnt_body(dst_hbm, zc_hbm, iota_hbm, ones_hbm,
              cnt_out,
              cnt_sh, vcnt, idxmat, didx, onesb, gsem, ssem):
  c = lax.axis_index("c")
  s = lax.axis_index("s")
  wid = c * NS + s

  pltpu.async_copy(iota_hbm.at[s], idxmat, ssem).wait()
  pltpu.async_copy(zc_hbm, vcnt, ssem).wait()
  pltpu.async_copy(ones_hbm, onesb, ssem).wait()

  for j in range(NZB):
    pltpu.async_copy(vcnt, cnt_sh.at[idxmat.at[j]], ssem).wait()
  plsc.subcore_barrier()

  def chunk(i, carry):
    base = wid * EPW + i * K
    pltpu.async_copy(dst_hbm.at[pl.ds(base, K)], didx, ssem).wait()
    pltpu.async_copy(onesb, cnt_sh.at[didx], ssem, add=True).wait()
    return carry

  lax.fori_loop(0, NCHUNK, chunk, 0)
  plsc.subcore_barrier()

  for j in range(NZB):
    pltpu.async_copy(cnt_sh.at[idxmat.at[j]], vcnt, gsem).wait()
    pltpu.async_copy(vcnt, cnt_out.at[c, pl.ds(s * RPT + j * ZB, ZB)],
                     ssem).wait()


_cnt = pl.kernel(
    _cnt_body,
    out_type=jax.ShapeDtypeStruct((NC, NP, F), jnp.float32),
    mesh=_MESH,
    scratch_types=[
        pltpu.VMEM_SHARED((NP, F), jnp.float32),
        pltpu.VMEM((ZB, F), jnp.float32),
        pltpu.VMEM((NZB, ZB), jnp.int32),
        pltpu.VMEM((K,), jnp.int32),
        pltpu.VMEM((K, F), jnp.float32),
        pltpu.SemaphoreType.DMA,
        pltpu.SemaphoreType.DMA,
    ],
    compiler_params=_PARAMS,
)

_DOT = dict(preferred_element_type=jnp.float32)


def _dense0_body(x, a0, a1, c0, c1, wr, wt, br, bt, wro, bro, out):
  cnt = c0[:, 0:1] + c1[:, 0:1]
  mean = (a0[...] + a1[...]) / jnp.maximum(cnt, 1.0)
  t = (jnp.dot(mean, wr[...], **_DOT) + jnp.dot(x[...], wt[...], **_DOT)
       + br[...] + bt[...])
  hc = jnp.maximum(t, 0.0)
  out[...] = jnp.maximum(jnp.dot(hc, wro[...], **_DOT) + bro[...], 0.0)


def _dense1_body(h, a0, a1, c0, c1, wr, wt, br, bt, wro, bro, wp, bp, out):
  cnt = c0[:, 0:1] + c1[:, 0:1]
  mean = (a0[...] + a1[...]) / jnp.maximum(cnt, 1.0)
  t = (jnp.dot(mean, wr[...], **_DOT) + jnp.dot(h[...], wt[...], **_DOT)
       + br[...] + bt[...])
  hc = jnp.maximum(t, 0.0)
  r = jnp.maximum(jnp.dot(hc, wro[...], **_DOT) + bro[...], 0.0)
  out[...] = jnp.dot(r, wp[...], **_DOT) + bp[...]


_row = pl.BlockSpec((BN, F), lambda i: (i, 0))
_cntb = pl.BlockSpec((BN, CW), lambda i: (i, 0))
_wmat = pl.BlockSpec((F, F), lambda i: (0, 0))
_bvec = pl.BlockSpec((1, F), lambda i: (0, 0))

_dense0 = pl.pallas_call(
    _dense0_body,
    grid=(N // BN,),
    in_specs=[_row, _row, _row, _cntb, _cntb, _wmat, _wmat, _bvec, _bvec,
              _wmat, _bvec],
    out_specs=_row,
    out_shape=jax.ShapeDtypeStruct((N, F), jnp.float32),
)

_dense1 = pl.pallas_call(
    _dense1_body,
    grid=(N // BN,),
    in_specs=[_row, _row, _row, _cntb, _cntb, _wmat, _wmat, _bvec, _bvec,
              _wmat, _bvec, _wmat, _bvec],
    out_specs=_row,
    out_shape=jax.ShapeDtypeStruct((N, F), jnp.float32),
)


def kernel(dynamic_node_features, edge_index, edge_weight,
           W_rel0, b_rel0, W_root0, b_root0, W_ro0, b_ro0,
           W_rel1, b_rel1, W_root1, b_root1, W_ro1, b_ro1,
           W_prd, b_prd):
  x = dynamic_node_features[:, 0, :]
  src = edge_index[0]
  dst = edge_index[1]
  ew = edge_weight
  zrow = jnp.zeros((ZB, F), jnp.float32)
  onesb = jnp.ones((K, F), jnp.float32)
  iota3 = jnp.arange(NP, dtype=jnp.int32).reshape(NS, NZB, ZB)

  bidx = jnp.broadcast_to(jnp.arange(16, dtype=jnp.int32)[:, None], (16, 16))
  cnt_p = _cnt(dst, zrow, iota3, onesb)
  acc_p = _agg(x, src, dst, ew, zrow, iota3, bidx)
  c0, c1 = cnt_p[0, :N, :CW], cnt_p[1, :N, :CW]
  h = _dense0(x, acc_p[0, :N], acc_p[1, :N], c0, c1,
              W_rel0.T, W_root0.T, b_rel0.reshape(1, F), b_root0.reshape(1, F),
              W_ro0.T, b_ro0.reshape(1, F))
  acc1_p = _agg(h, src, dst, ew, zrow, iota3, bidx)
  wp_pad = jnp.pad(W_prd.T, ((0, 0), (0, F - 1)))
  bp_pad = jnp.pad(b_prd.reshape(1, 1), ((0, 0), (0, F - 1)))
  xh = _dense1(h, acc1_p[0, :N], acc1_p[1, :N], c0, c1,
               W_rel1.T, W_root1.T, b_rel1.reshape(1, F), b_root1.reshape(1, F),
               W_ro1.T, b_ro1.reshape(1, F), wp_pad, bp_pad)
  return xh[:, :1].reshape(N, 1, 1)


# speedup vs baseline: 3.6761x; 1.0033x over previous
import jax
import jax.numpy as jnp
from jax import lax
from jax.experimental import pallas as pl
from jax.experimental.pallas import tpu as pltpu
from jax.experimental.pallas import tpu_sc as plsc

N = 10000
E = 320000
F = 128
NC = 2
NS = 16
NW = NC * NS
EPW = E // NW
K = 80
NCHUNK = EPW // K
NP = 10240
RPT = NP // NS
ZB = 128
NZB = RPT // ZB
CW = 16
BN = 1000

_MESH = plsc.VectorSubcoreMesh(
    core_axis_name="c", subcore_axis_name="s", num_cores=NC, num_subcores=NS)
_PARAMS = pltpu.CompilerParams(needs_layout_passes=False)


def _agg_body(x_hbm, src_hbm, dst_hbm, w_hbm, z_hbm, iota_hbm, bidx_hbm,
              acc_out,
              acc_sh, vbuf, idxmat, bidx,
              rows0, sidx0, didx0, wbuf0, rows1, sidx1, didx1, wbuf1,
              ssem, gsem0, gsem1, ssm0, ssm1,
              is00, is01, is02, is10, is11, is12):
  c = lax.axis_index("c")
  s = lax.axis_index("s")
  wid = c * NS + s
  ROWS = (rows0, rows1)
  SIDX = (sidx0, sidx1)
  DIDX = (didx0, didx1)
  WBUF = (wbuf0, wbuf1)
  GSEM = (gsem0, gsem1)
  SSEM = (ssm0, ssm1)
  ISEM = ((is00, is01, is02), (is10, is11, is12))

  pltpu.async_copy(iota_hbm.at[s], idxmat, ssem).wait()
  pltpu.async_copy(bidx_hbm, bidx, ssem).wait()
  pltpu.async_copy(z_hbm, vbuf, ssem).wait()

  for j in range(NZB):
    pltpu.async_copy(vbuf, acc_sh.at[idxmat.at[j]], ssem).wait()
  plsc.subcore_barrier()

  def scale_rows(rows_ref, wbuf_ref):
    for g in range(K // 16):
      for j in range(16):
        k = g * 16 + j
        wb = plsc.load_gather(wbuf_ref, [bidx[j, :] + (g * 16)])
        for f in range(F // 16):
          sl = pl.ds(f * 16, 16)
          rows_ref[k, sl] = rows_ref[k, sl] * wb

  def start_idx(ii, b):
    base = wid * EPW + ii * K
    return (pltpu.async_copy(src_hbm.at[pl.ds(base, K)], SIDX[b], ISEM[b][0]),
            pltpu.async_copy(dst_hbm.at[pl.ds(base, K)], DIDX[b], ISEM[b][1]),
            pltpu.async_copy(w_hbm.at[pl.ds(base, K)], WBUF[b], ISEM[b][2]))

  def chunk_pair(t, carry):
    ia = 2 * t
    descs = [start_idx(ia, 0), start_idx(ia + 1, 1)]
    gath = []
    for b in (0, 1):
      descs[b][0].wait()
      gath.append(pltpu.async_copy(x_hbm.at[SIDX[b]], ROWS[b], GSEM[b]))
    scat = []
    for b in (0, 1):
      descs[b][2].wait()
      gath[b].wait()
      scale_rows(ROWS[b], WBUF[b])
      descs[b][1].wait()
      scat.append(pltpu.async_copy(ROWS[b], acc_sh.at[DIDX[b]], SSEM[b],
                                   add=True))
    scat[0].wait()
    scat[1].wait()
    return carry

  lax.fori_loop(0, NCHUNK // 2, chunk_pair, 0)
  if NCHUNK % 2:
    d = start_idx(NCHUNK - 1, 0)
    d[0].wait()
    g = pltpu.async_copy(x_hbm.at[sidx0], rows0, gsem0)
    d[2].wait()
    g.wait()
    scale_rows(rows0, wbuf0)
    d[1].wait()
    pltpu.async_copy(rows0, acc_sh.at[didx0], ssm0, add=True).wait()
  plsc.subcore_barrier()

  for j in range(NZB):
    pltpu.async_copy(acc_sh.at[idxmat.at[j]], vbuf, gsem0).wait()
    pltpu.async_copy(vbuf, acc_out.at[c, pl.ds(s * RPT + j * ZB, ZB)],
                     ssem).wait()


_agg = pl.kernel(
    _agg_body,
    out_type=jax.ShapeDtypeStruct((NC, NP, F), jnp.float32),
    mesh=_MESH,
    scratch_types=[
        pltpu.VMEM_SHARED((NP, F), jnp.float32),
        pltpu.VMEM((ZB, F), jnp.float32),
        pltpu.VMEM((NZB, ZB), jnp.int32),
        pltpu.VMEM((16, 16), jnp.int32),
        pltpu.VMEM((K, F), jnp.float32),
        pltpu.VMEM((K,), jnp.int32),
        pltpu.VMEM((K,), jnp.int32),
        pltpu.VMEM((K,), jnp.float32),
        pltpu.VMEM((K, F), jnp.float32),
        pltpu.VMEM((K,), jnp.int32),
        pltpu.VMEM((K,), jnp.int32),
        pltpu.VMEM((K,), jnp.float32),
        pltpu.SemaphoreType.DMA,
        pltpu.SemaphoreType.DMA,
        pltpu.SemaphoreType.DMA,
        pltpu.SemaphoreType.DMA,
        pltpu.SemaphoreType.DMA,
        pltpu.SemaphoreType.DMA,
        pltpu.SemaphoreType.DMA,
        pltpu.SemaphoreType.DMA,
        pltpu.SemaphoreType.DMA,
        pltpu.SemaphoreType.DMA,
        pltpu.SemaphoreType.DMA,
    ],
    compiler_params=_PARAMS,
)


CWIDE = F


def _cnt_body(dst_hbm, zc_hbm, iota_hbm, ones_hbm,
              cnt_out,
              cnt_sh, vcnt, idxmat, didx, onesb, gsem, ssem):
  c = lax.axis_index("c")
  s = lax.axis_index("s")
  wid = c * NS + s

  pltpu.async_copy(iota_hbm.at[s], idxmat, ssem).wait()
  pltpu.async_copy(zc_hbm, vcnt, ssem).wait()
  pltpu.async_copy(ones_hbm, onesb, ssem).wait()

  for j in range(NZB):
    pltpu.async_copy(vcnt, cnt_sh.at[idxmat.at[j]], ssem).wait()
  plsc.subcore_barrier()

  def chunk(i, carry):
    base = wid * EPW + i * K
    pltpu.async_copy(dst_hbm.at[pl.ds(base, K)], didx, ssem).wait()
    pltpu.async_copy(onesb, cnt_sh.at[didx], ssem, add=True).wait()
    return carry

  lax.fori_loop(0, NCHUNK, chunk, 0)
  plsc.subcore_barrier()

  for j in range(NZB):
    pltpu.async_copy(cnt_sh.at[idxmat.at[j]], vcnt, gsem).wait()
    pltpu.async_copy(vcnt, cnt_out.at[c, pl.ds(s * RPT + j * ZB, ZB)],
                     ssem).wait()


_cnt = pl.kernel(
    _cnt_body,
    out_type=jax.ShapeDtypeStruct((NC, NP, CWIDE), jnp.float32),
    mesh=_MESH,
    scratch_types=[
        pltpu.VMEM_SHARED((NP, CWIDE), jnp.float32),
        pltpu.VMEM((ZB, CWIDE), jnp.float32),
        pltpu.VMEM((NZB, ZB), jnp.int32),
        pltpu.VMEM((K,), jnp.int32),
        pltpu.VMEM((K, CWIDE), jnp.float32),
        pltpu.SemaphoreType.DMA,
        pltpu.SemaphoreType.DMA,
    ],
    compiler_params=_PARAMS,
)

_DOT = dict(preferred_element_type=jnp.float32)


def _dense0_body(x, a0, a1, c0, c1, wr, wt, br, bt, wro, bro, out):
  cnt = c0[:, 0:1] + c1[:, 0:1]
  mean = (a0[...] + a1[...]) / jnp.maximum(cnt, 1.0)
  t = (jnp.dot(mean, wr[...], **_DOT) + jnp.dot(x[...], wt[...], **_DOT)
       + br[...] + bt[...])
  hc = jnp.maximum(t, 0.0)
  out[...] = jnp.maximum(jnp.dot(hc, wro[...], **_DOT) + bro[...], 0.0)


def _dense1_body(h, a0, a1, c0, c1, wr, wt, br, bt, wro, bro, wp, bp, out):
  cnt = c0[:, 0:1] + c1[:, 0:1]
  mean = (a0[...] + a1[...]) / jnp.maximum(cnt, 1.0)
  t = (jnp.dot(mean, wr[...], **_DOT) + jnp.dot(h[...], wt[...], **_DOT)
       + br[...] + bt[...])
  hc = jnp.maximum(t, 0.0)
  r = jnp.maximum(jnp.dot(hc, wro[...], **_DOT) + bro[...], 0.0)
  out[...] = jnp.dot(r, wp[...], **_DOT) + bp[...]


_row = pl.BlockSpec((BN, F), lambda i: (i, 0))
_cntb = pl.BlockSpec((BN, CW), lambda i: (i, 0))
_wmat = pl.BlockSpec((F, F), lambda i: (0, 0))
_bvec = pl.BlockSpec((1, F), lambda i: (0, 0))

_dense0 = pl.pallas_call(
    _dense0_body,
    grid=(N // BN,),
    in_specs=[_row, _row, _row, _cntb, _cntb, _wmat, _wmat, _bvec, _bvec,
              _wmat, _bvec],
    out_specs=_row,
    out_shape=jax.ShapeDtypeStruct((N, F), jnp.float32),
)

_dense1 = pl.pallas_call(
    _dense1_body,
    grid=(N // BN,),
    in_specs=[_row, _row, _row, _cntb, _cntb, _wmat, _wmat, _bvec, _bvec,
              _wmat, _bvec, _wmat, _bvec],
    out_specs=_row,
    out_shape=jax.ShapeDtypeStruct((N, F), jnp.float32),
)


def kernel(dynamic_node_features, edge_index, edge_weight,
           W_rel0, b_rel0, W_root0, b_root0, W_ro0, b_ro0,
           W_rel1, b_rel1, W_root1, b_root1, W_ro1, b_ro1,
           W_prd, b_prd):
  x = dynamic_node_features[:, 0, :]
  src = edge_index[0]
  dst = edge_index[1]
  ew = edge_weight
  zrow = jnp.zeros((ZB, F), jnp.float32)
  onesb = jnp.ones((K, CWIDE), jnp.float32)
  zcnt = jnp.zeros((ZB, CWIDE), jnp.float32)
  iota3 = jnp.arange(NP, dtype=jnp.int32).reshape(NS, NZB, ZB)

  bidx = jnp.broadcast_to(jnp.arange(16, dtype=jnp.int32)[:, None], (16, 16))
  cnt_p = _cnt(dst, zcnt, iota3, onesb)
  acc_p = _agg(x, src, dst, ew, zrow, iota3, bidx)
  c0, c1 = cnt_p[0, :N, :CW], cnt_p[1, :N, :CW]
  h = _dense0(x, acc_p[0, :N], acc_p[1, :N], c0, c1,
              W_rel0.T, W_root0.T, b_rel0.reshape(1, F), b_root0.reshape(1, F),
              W_ro0.T, b_ro0.reshape(1, F))
  acc1_p = _agg(h, src, dst, ew, zrow, iota3, bidx)
  wp_pad = jnp.pad(W_prd.T, ((0, 0), (0, F - 1)))
  bp_pad = jnp.pad(b_prd.reshape(1, 1), ((0, 0), (0, F - 1)))
  xh = _dense1(h, acc1_p[0, :N], acc1_p[1, :N], c0, c1,
               W_rel1.T, W_root1.T, b_rel1.reshape(1, F), b_root1.reshape(1, F),
               W_ro1.T, b_ro1.reshape(1, F), wp_pad, bp_pad)
  return xh[:, :1].reshape(N, 1, 1)
